# Initial kernel scaffold; baseline (speedup 1.0000x reference)
import jax
import jax.numpy as jnp
from jax.experimental import pallas as pl


def kernel(t, embeddings):
    raise NotImplementedError("write your pallas kernel here")


# speedup vs baseline: 1.1065x; 1.1065x over previous
import functools

import jax
import jax.numpy as jnp
from jax import lax
from jax.experimental import pallas as pl
from jax.experimental.pallas import tpu as pltpu
from jax.experimental.pallas import tpu_sc as plsc

EMB = 32

_info = plsc.get_sparse_core_info()
_NC, _NS = _info.num_cores, _info.num_subcores
_NW = _NC * _NS


@functools.cache
def _make_gather(n_rows, b_per_w, n_chunks, chunk):
    mesh = plsc.VectorSubcoreMesh(core_axis_name="c", subcore_axis_name="s")

    @functools.partial(
        pl.kernel,
        mesh=mesh,
        out_type=jax.ShapeDtypeStruct((_NW * b_per_w, EMB), jnp.float32),
        scratch_types=[
            pltpu.VMEM((n_chunks, chunk), jnp.int32),
            pltpu.VMEM((chunk, EMB), jnp.float32),
            pltpu.SemaphoreType.DMA,
        ],
        compiler_params=pltpu.CompilerParams(use_tc_tiling_on_sc=False),
    )
    def gather(t_hbm, table_hbm, out_hbm, idx_v, rows_v, sem):
        wid = lax.axis_index("s") * _NC + lax.axis_index("c")
        base = wid * b_per_w
        pltpu.sync_copy(t_hbm.at[wid], idx_v)
        for i in range(n_chunks):
            pltpu.async_copy(table_hbm.at[idx_v.at[i]], rows_v, sem).wait()
            pltpu.sync_copy(rows_v, out_hbm.at[pl.ds(base + i * chunk, chunk)])

    return gather


def kernel(t, embeddings):
    b_per_w = t.size // _NW
    n_chunks = 16
    chunk = b_per_w // n_chunks
    tf = t.reshape(_NW, n_chunks, chunk)
    fn = _make_gather(embeddings.shape[0], b_per_w, n_chunks, chunk)
    out = fn(tf, embeddings)
    return out.reshape(t.shape + (EMB,))


# speedup vs baseline: 1.1124x; 1.0053x over previous
import functools

import jax
import jax.numpy as jnp
from jax import lax
from jax.experimental import pallas as pl
from jax.experimental.pallas import tpu as pltpu
from jax.experimental.pallas import tpu_sc as plsc

EMB = 32
NBUF = 2

_info = plsc.get_sparse_core_info()
_NC, _NS = _info.num_cores, _info.num_subcores
_NW = _NC * _NS


@functools.cache
def _make_gather(n_rows, b_per_w, n_chunks, chunk):
    mesh = plsc.VectorSubcoreMesh(core_axis_name="c", subcore_axis_name="s")
    scratch = (
        [pltpu.VMEM((n_chunks, chunk), jnp.int32)]
        + [pltpu.VMEM((chunk, EMB), jnp.float32) for _ in range(NBUF)]
        + [pltpu.SemaphoreType.DMA for _ in range(2 * NBUF)]
    )

    @functools.partial(
        pl.kernel,
        mesh=mesh,
        out_type=jax.ShapeDtypeStruct((_NW * b_per_w, EMB), jnp.float32),
        scratch_types=scratch,
        compiler_params=pltpu.CompilerParams(use_tc_tiling_on_sc=False),
    )
    def gather(t_hbm, table_hbm, out_hbm, idx_v, *bufs_and_sems):
        rows = bufs_and_sems[:NBUF]
        gs = bufs_and_sems[NBUF : 2 * NBUF]
        ws = bufs_and_sems[2 * NBUF :]
        wid = lax.axis_index("s") * _NC + lax.axis_index("c")
        base = wid * b_per_w
        pltpu.sync_copy(t_hbm.at[wid], idx_v)

        gcp = [None] * NBUF
        wcp = [None] * NBUF

        def start_write(i):
            b = i % NBUF
            gcp[b].wait()
            wcp[b] = pltpu.async_copy(
                rows[b], out_hbm.at[pl.ds(base + i * chunk, chunk)], ws[b]
            )

        for i in range(n_chunks):
            b = i % NBUF
            if wcp[b] is not None:
                wcp[b].wait()
            gcp[b] = pltpu.async_copy(table_hbm.at[idx_v.at[i]], rows[b], gs[b])
            if i >= NBUF - 1:
                start_write(i - (NBUF - 1))
        for i in range(max(0, n_chunks - (NBUF - 1)), n_chunks):
            start_write(i)
        for w in wcp:
            if w is not None:
                w.wait()

    return gather


def kernel(t, embeddings):
    b_per_w = t.size // _NW
    n_chunks = 20
    chunk = b_per_w // n_chunks
    tf = t.reshape(_NW, n_chunks, chunk)
    fn = _make_gather(embeddings.shape[0], b_per_w, n_chunks, chunk)
    out = fn(tf, embeddings)
    return out.reshape(t.shape + (EMB,))
